# baseline (speedup 1.0000x reference)
import functools

import jax
import jax.numpy as jnp
from jax import lax
from jax.experimental import pallas as pl
from jax.experimental.pallas import tpu as pltpu
from jax.experimental.pallas import tpu_sc as plsc

NUM_INPUTS = 32
NUM_GROUPS = 1000000
BATCH = 16384
NC = 2
NS = 16
NW = NC * NS
BPW = BATCH // NW
GRP = 16
NGRP = BPW // GRP
LANE = 128


def _sc_body(xt_ref, ids_ref, wb_ref, tab_ref, out_ref,
             ids_v, xt_v, cols_v, out_v, wb_v, sem):
    wid = lax.axis_index("s") * NC + lax.axis_index("c")
    base = wid * BPW

    pltpu.sync_copy(ids_ref.at[pl.ds(base, BPW)], ids_v)
    pltpu.sync_copy(xt_ref.at[:, pl.ds(base, BPW)], xt_v)
    pltpu.sync_copy(wb_ref, wb_v)

    lanes = lax.iota(jnp.int32, GRP)

    def group(g, _):
        o = g * GRP
        idv = ids_v[pl.ds(o, GRP)]
        colv = idv // LANE
        loff = idv - colv * LANE

        copies = []
        for k in range(GRP):
            off = pl.multiple_of(colv[k] * LANE, LANE)
            copies.append(pltpu.async_copy(
                tab_ref.at[:, pl.ds(off, LANE)],
                cols_v.at[k],
                sem))
        for cp in copies:
            cp.wait()

        acc = jnp.zeros((GRP,), jnp.float32)
        for j in range(NUM_INPUTS):
            wv = plsc.load_gather(
                cols_v, [lanes, jnp.full((GRP,), j, jnp.int32), loff])
            xv = xt_v[j, pl.ds(o, GRP)]
            acc = acc + xv * (wv + wb_v[j, 0:GRP])
        out_v[pl.ds(o, GRP)] = acc
        return 0

    lax.fori_loop(0, NGRP, group, 0)

    pltpu.sync_copy(out_v, out_ref.at[pl.ds(base, BPW)])


@jax.jit
def _run(XT, ids, wb, tabT):
    mesh = plsc.VectorSubcoreMesh(core_axis_name="c", subcore_axis_name="s")
    f = functools.partial(
        pl.kernel,
        out_type=jax.ShapeDtypeStruct((BATCH,), jnp.float32),
        mesh=mesh,
        compiler_params=pltpu.CompilerParams(needs_layout_passes=False),
        scratch_types=[
            pltpu.VMEM((BPW,), jnp.int32),
            pltpu.VMEM((NUM_INPUTS, BPW), jnp.float32),
            pltpu.VMEM((GRP, NUM_INPUTS, LANE), jnp.float32),
            pltpu.VMEM((BPW,), jnp.float32),
            pltpu.VMEM((NUM_INPUTS, GRP), jnp.float32),
            pltpu.SemaphoreType.DMA,
        ],
    )(_sc_body)
    return f(XT, ids, wb, tabT)


def kernel(X, ids, W_weight, W_random):
    ids = ids.astype(jnp.int32)
    XT = jnp.transpose(X)
    tabT = jnp.transpose(W_random)
    wb = jnp.broadcast_to(W_weight.reshape(NUM_INPUTS, 1), (NUM_INPUTS, GRP))
    return _run(XT, ids, wb, tabT)
